# baseline (speedup 1.0000x reference)
import numpy as np
import jax
import jax.numpy as jnp
from jax.experimental import pallas as pl
from jax.experimental.pallas import tpu as pltpu


def _toeplitz(w_hwio, win, order, ncols, col0):
    k, _, cin, cout = w_hwio.shape
    nwo = len(order)
    s = np.zeros((k, win, nwo), np.float32)
    for kw in range(k):
        for col, wo in enumerate(order):
            s[kw, wo + kw, col] = 1.0
    t = jnp.einsum("kxw,hkio->hxiwo", jnp.asarray(s), w_hwio)
    t = t.reshape(k, win * cin, nwo * cout)
    return jnp.zeros((k, win * cin, ncols),
                     jnp.float32).at[:, :, col0:col0 + nwo * cout].set(t)


def _dot(lhs, rhs):
    return jax.lax.dot_general(lhs, rhs, (((lhs.ndim - 1,), (0,)), ((), ())),
                               preferred_element_type=jnp.float32)


def _fused_kernel(x_ref, t1a_ref, t1b_ref, t2_ref, c2b_ref,
                  t3_ref, c3b_ref, w1_ref, b1_ref, g_ref, s_ref,
                  w2_ref, b2_ref, o_ref):
    x = x_ref[...]
    xa, xb = x[:, 0:6, :], x[:, 1:7, :]

    act1 = [jnp.maximum(_dot(xa, t1a_ref[c]) + _dot(xb, t1b_ref[c]), 0.0)
            .astype(jnp.bfloat16) for c in range(4)]

    a2 = [
        jnp.pad(act1[2], ((0, 0), (1, 0), (0, 0))),
        jnp.pad(act1[3], ((0, 0), (1, 0), (0, 0))),
        jnp.pad(act1[0], ((0, 0), (0, 1), (0, 0))),
        jnp.pad(act1[1], ((0, 0), (0, 1), (0, 0))),
    ]

    act2 = []
    for p in range(2):
        hacc = []
        for hh in range(2):
            acc = None
            for kh in range(5):
                r = 2 * p + hh + kh
                q, s0 = r % 4, r // 4
                h = _dot(a2[q][:, s0:s0 + 6, :], t2_ref[kh])
                acc = h if acc is None else acc + h
            hacc.append(acc)
        z = jnp.maximum(jnp.maximum(hacc[0], hacc[1]) + c2b_ref[...], 0.0)
        act2.append(jnp.maximum(z[:, :, :96], z[:, :, 96:])
                    .astype(jnp.bfloat16))

    hacc3 = []
    for hh in range(2):
        acc = None
        for kh in range(5):
            r = hh + kh
            q, s0 = r % 2, r // 2
            h = _dot(act2[q][:, s0:s0 + 4, :], t3_ref[kh])
            acc = h if acc is None else acc + h
        hacc3.append(acc)
    z3 = jnp.maximum(jnp.maximum(hacc3[0], hacc3[1]) + c3b_ref[...], 0.0)
    act3 = jnp.maximum(z3[:, :, :64], z3[:, :, 64:]).astype(jnp.bfloat16)

    acc = None
    for h4 in range(4):
        h = _dot(act3[:, h4, :], w1_ref[h4])
        acc = h if acc is None else acc + h
    h = jnp.maximum(acc + b1_ref[...], 0.0)
    h = jnp.maximum(h * g_ref[...] + s_ref[...], 0.0)
    zz = _dot(h.astype(jnp.bfloat16), w2_ref[...]) + b2_ref[...]
    m = jnp.max(zz, axis=1, keepdims=True)
    sz = zz - m
    lse = jnp.log(jnp.sum(jnp.exp(sz), axis=1, keepdims=True))
    o_ref[...] = (sz - lse).astype(o_ref.dtype)


@jax.jit
def _forward(x, conv1_w, conv1_b, conv2_w, conv2_b, conv3_w, conv3_b,
             fc1_w, fc1_b, fc2_w, fc2_b, bn_gamma, bn_beta, bn_mean, bn_var):
    b = x.shape[0]
    x7 = jnp.concatenate([x.reshape(b, 7, 112),
                          jnp.ones((b, 7, 1), jnp.float32)],
                         axis=2).astype(jnp.bfloat16)

    sa = np.zeros((4, 5, 5, 113, 24), np.float32)
    sb = np.zeros_like(sa)
    for c in range(4):
        for kh in range(5):
            tgt, q = (sa, c + kh) if c + kh < 4 else (sb, c + kh - 4)
            for kw in range(5):
                for wo in range(24):
                    tgt[c, kh, kw, q * 28 + wo + kw, wo] = 1.0
    w1sq = conv1_w[:, :, 0, :]
    t1a = jnp.einsum("chkxw,hko->cxwo", jnp.asarray(sa),
                     w1sq).reshape(4, 113, 192)
    t1b = jnp.einsum("chkxw,hko->cxwo", jnp.asarray(sb),
                     w1sq).reshape(4, 113, 192)
    t1a = t1a.at[:, 112, :].set(jnp.tile(conv1_b, 24))
    zpad16 = jnp.zeros((4, 113, 16), jnp.float32)
    t1a = jnp.concatenate([zpad16, t1a, zpad16],
                          axis=2).astype(jnp.bfloat16)
    t1b = jnp.concatenate([zpad16, t1b, zpad16],
                          axis=2).astype(jnp.bfloat16)

    ev24 = list(range(0, 24, 2))
    od24 = list(range(1, 24, 2))
    t2 = (_toeplitz(conv2_w, 28, ev24, 192, 0)
          + _toeplitz(conv2_w, 28, od24, 192, 96)
          ).astype(jnp.bfloat16)
    t3 = (_toeplitz(conv3_w, 12, [0, 2, 4, 6], 128, 0)
          + _toeplitz(conv3_w, 12, [1, 3, 5, 7], 128, 64)
          ).astype(jnp.bfloat16)

    c2b = jnp.concatenate([jnp.tile(conv2_b, 12)] * 2).reshape(1, 1, 192)
    c3b = jnp.concatenate([jnp.tile(conv3_b, 4)] * 2).reshape(1, 1, 128)

    w1 = (fc1_w.reshape(16, 4, 4, 64).transpose(1, 2, 0, 3)
          .reshape(4, 64, 64).astype(jnp.bfloat16))
    scale = bn_gamma * jax.lax.rsqrt(bn_var + 1e-5)
    shift = bn_beta - bn_mean * scale
    w2p = (jnp.zeros((64, 128), jnp.float32).at[:, :10].set(fc2_w)
           .astype(jnp.bfloat16))
    b2p = jnp.full((1, 128), -1e30, jnp.float32).at[0, :10].set(fc2_b)

    tb = 128 if b % 128 == 0 else b
    flops = 2 * b * (6 * 2 * 4 * 112 * 192 + 6 * 4 * 5 * 224 * 192
                     + 4 * 2 * 5 * 96 * 128 + 4 * 64 * 64 + 64 * 128)
    out = pl.pallas_call(
        _fused_kernel,
        out_shape=jax.ShapeDtypeStruct((b, 128), jnp.float32),
        grid_spec=pltpu.PrefetchScalarGridSpec(
            num_scalar_prefetch=0,
            grid=(b // tb,),
            in_specs=[
                pl.BlockSpec((tb, 7, 113), lambda i: (i, 0, 0)),
                pl.BlockSpec((4, 113, 224), lambda i: (0, 0, 0)),
                pl.BlockSpec((4, 113, 224), lambda i: (0, 0, 0)),
                pl.BlockSpec((5, 224, 192), lambda i: (0, 0, 0)),
                pl.BlockSpec((1, 1, 192), lambda i: (0, 0, 0)),
                pl.BlockSpec((5, 96, 128), lambda i: (0, 0, 0)),
                pl.BlockSpec((1, 1, 128), lambda i: (0, 0, 0)),
                pl.BlockSpec((4, 64, 64), lambda i: (0, 0, 0)),
                pl.BlockSpec((1, 64), lambda i: (0, 0)),
                pl.BlockSpec((1, 64), lambda i: (0, 0)),
                pl.BlockSpec((1, 64), lambda i: (0, 0)),
                pl.BlockSpec((64, 128), lambda i: (0, 0)),
                pl.BlockSpec((1, 128), lambda i: (0, 0)),
            ],
            out_specs=pl.BlockSpec((tb, 128), lambda i: (i, 0)),
        ),
        compiler_params=pltpu.CompilerParams(
            dimension_semantics=("parallel",)),
        cost_estimate=pl.CostEstimate(
            flops=int(flops), transcendentals=int(b * 128),
            bytes_accessed=int(x.size * 4 + b * 128 * 4)),
    )(x7, t1a, t1b, t2, c2b, t3, c3b, w1,
      fc1_b.reshape(1, 64), scale.reshape(1, 64), shift.reshape(1, 64),
      w2p, b2p)
    return out[:, :10]


def kernel(x, conv1_w, conv1_b, conv2_w, conv2_b, conv3_w, conv3_b,
           fc1_w, fc1_b, fc2_w, fc2_b, bn_gamma, bn_beta, bn_mean, bn_var):
    return _forward(x, conv1_w, conv1_b, conv2_w, conv2_b, conv3_w, conv3_b,
                    fc1_w, fc1_b, fc2_w, fc2_b, bn_gamma, bn_beta,
                    bn_mean, bn_var)


# speedup vs baseline: 1.1606x; 1.1606x over previous
import numpy as np
import jax
import jax.numpy as jnp
from jax.experimental import pallas as pl
from jax.experimental.pallas import tpu as pltpu


def _toeplitz(w_hwio, win, order, ncols, col0):
    k, _, cin, cout = w_hwio.shape
    nwo = len(order)
    s = np.zeros((k, win, nwo), np.float32)
    for kw in range(k):
        for col, wo in enumerate(order):
            s[kw, wo + kw, col] = 1.0
    t = jnp.einsum("kxw,hkio->hxiwo", jnp.asarray(s), w_hwio)
    t = t.reshape(k, win * cin, nwo * cout)
    return jnp.zeros((k, win * cin, ncols),
                     jnp.float32).at[:, :, col0:col0 + nwo * cout].set(t)


def _dot(lhs, rhs):
    return jax.lax.dot_general(lhs, rhs, (((lhs.ndim - 1,), (0,)), ((), ())),
                               preferred_element_type=jnp.float32)


def _fused_kernel(x_ref, t1a_ref, t1b_ref, t2_ref, c2b_ref,
                  t3_ref, c3b_ref, w1_ref, b1_ref, g_ref, s_ref,
                  w2_ref, b2_ref, o_ref):
    x = x_ref[...]
    xa, xb = x[:, 0:6, :], x[:, 1:7, :]

    act1 = [jnp.maximum(_dot(xa, t1a_ref[c]) + _dot(xb, t1b_ref[c]), 0.0)
             for c in range(4)]

    a2 = [
        jnp.pad(act1[2], ((0, 0), (1, 0), (0, 0))),
        jnp.pad(act1[3], ((0, 0), (1, 0), (0, 0))),
        jnp.pad(act1[0], ((0, 0), (0, 1), (0, 0))),
        jnp.pad(act1[1], ((0, 0), (0, 1), (0, 0))),
    ]

    act2 = []
    for p in range(2):
        hacc = []
        for hh in range(2):
            acc = None
            for kh in range(5):
                r = 2 * p + hh + kh
                q, s0 = r % 4, r // 4
                h = _dot(a2[q][:, s0:s0 + 6, :], t2_ref[kh])
                acc = h if acc is None else acc + h
            hacc.append(acc)
        z = jnp.maximum(jnp.maximum(hacc[0], hacc[1]) + c2b_ref[...], 0.0)
        act2.append(jnp.maximum(z[:, :, :96], z[:, :, 96:]))

    hacc3 = []
    for hh in range(2):
        acc = None
        for kh in range(5):
            r = hh + kh
            q, s0 = r % 2, r // 2
            h = _dot(act2[q][:, s0:s0 + 4, :], t3_ref[kh])
            acc = h if acc is None else acc + h
        hacc3.append(acc)
    z3 = jnp.maximum(jnp.maximum(hacc3[0], hacc3[1]) + c3b_ref[...], 0.0)
    act3 = jnp.maximum(z3[:, :, :64], z3[:, :, 64:])

    acc = None
    for h4 in range(4):
        h = _dot(act3[:, h4, :], w1_ref[h4])
        acc = h if acc is None else acc + h
    h = jnp.maximum(acc + b1_ref[...], 0.0)
    h = jnp.maximum(h * g_ref[...] + s_ref[...], 0.0)
    zz = _dot(h, w2_ref[...]) + b2_ref[...]
    m = jnp.max(zz, axis=1, keepdims=True)
    sz = zz - m
    lse = jnp.log(jnp.sum(jnp.exp(sz), axis=1, keepdims=True))
    o_ref[...] = (sz - lse).astype(o_ref.dtype)


@jax.jit
def _forward(x, conv1_w, conv1_b, conv2_w, conv2_b, conv3_w, conv3_b,
             fc1_w, fc1_b, fc2_w, fc2_b, bn_gamma, bn_beta, bn_mean, bn_var):
    b = x.shape[0]
    x7 = jnp.concatenate([x.reshape(b, 7, 112),
                          jnp.ones((b, 7, 1), jnp.float32)], axis=2)

    sa = np.zeros((4, 5, 5, 113, 24), np.float32)
    sb = np.zeros_like(sa)
    for c in range(4):
        for kh in range(5):
            tgt, q = (sa, c + kh) if c + kh < 4 else (sb, c + kh - 4)
            for kw in range(5):
                for wo in range(24):
                    tgt[c, kh, kw, q * 28 + wo + kw, wo] = 1.0
    w1sq = conv1_w[:, :, 0, :]
    t1a = jnp.einsum("chkxw,hko->cxwo", jnp.asarray(sa),
                     w1sq).reshape(4, 113, 192)
    t1b = jnp.einsum("chkxw,hko->cxwo", jnp.asarray(sb),
                     w1sq).reshape(4, 113, 192)
    t1a = t1a.at[:, 112, :].set(jnp.tile(conv1_b, 24))
    zpad16 = jnp.zeros((4, 113, 16), jnp.float32)
    t1a = jnp.concatenate([zpad16, t1a, zpad16], axis=2)
    t1b = jnp.concatenate([zpad16, t1b, zpad16], axis=2)

    ev24 = list(range(0, 24, 2))
    od24 = list(range(1, 24, 2))
    t2 = (_toeplitz(conv2_w, 28, ev24, 192, 0)
          + _toeplitz(conv2_w, 28, od24, 192, 96))
    t3 = (_toeplitz(conv3_w, 12, [0, 2, 4, 6], 128, 0)
          + _toeplitz(conv3_w, 12, [1, 3, 5, 7], 128, 64))

    c2b = jnp.concatenate([jnp.tile(conv2_b, 12)] * 2).reshape(1, 1, 192)
    c3b = jnp.concatenate([jnp.tile(conv3_b, 4)] * 2).reshape(1, 1, 128)

    w1 = fc1_w.reshape(16, 4, 4, 64).transpose(1, 2, 0, 3).reshape(4, 64, 64)
    scale = bn_gamma * jax.lax.rsqrt(bn_var + 1e-5)
    shift = bn_beta - bn_mean * scale
    w2p = jnp.zeros((64, 128), jnp.float32).at[:, :10].set(fc2_w)
    b2p = jnp.full((1, 128), -1e30, jnp.float32).at[0, :10].set(fc2_b)

    tb = 128 if b % 128 == 0 else b
    flops = 2 * b * (6 * 2 * 4 * 112 * 192 + 6 * 4 * 5 * 224 * 192
                     + 4 * 2 * 5 * 96 * 128 + 4 * 64 * 64 + 64 * 128)
    out = pl.pallas_call(
        _fused_kernel,
        out_shape=jax.ShapeDtypeStruct((b, 128), jnp.float32),
        grid_spec=pltpu.PrefetchScalarGridSpec(
            num_scalar_prefetch=0,
            grid=(b // tb,),
            in_specs=[
                pl.BlockSpec((tb, 7, 113), lambda i: (i, 0, 0)),
                pl.BlockSpec((4, 113, 224), lambda i: (0, 0, 0)),
                pl.BlockSpec((4, 113, 224), lambda i: (0, 0, 0)),
                pl.BlockSpec((5, 224, 192), lambda i: (0, 0, 0)),
                pl.BlockSpec((1, 1, 192), lambda i: (0, 0, 0)),
                pl.BlockSpec((5, 96, 128), lambda i: (0, 0, 0)),
                pl.BlockSpec((1, 1, 128), lambda i: (0, 0, 0)),
                pl.BlockSpec((4, 64, 64), lambda i: (0, 0, 0)),
                pl.BlockSpec((1, 64), lambda i: (0, 0)),
                pl.BlockSpec((1, 64), lambda i: (0, 0)),
                pl.BlockSpec((1, 64), lambda i: (0, 0)),
                pl.BlockSpec((64, 128), lambda i: (0, 0)),
                pl.BlockSpec((1, 128), lambda i: (0, 0)),
            ],
            out_specs=pl.BlockSpec((tb, 128), lambda i: (i, 0)),
        ),
        compiler_params=pltpu.CompilerParams(
            dimension_semantics=("parallel",)),
        cost_estimate=pl.CostEstimate(
            flops=int(flops), transcendentals=int(b * 128),
            bytes_accessed=int(x.size * 4 + b * 128 * 4)),
    )(x7, t1a, t1b, t2, c2b, t3, c3b, w1,
      fc1_b.reshape(1, 64), scale.reshape(1, 64), shift.reshape(1, 64),
      w2p, b2p)
    return out[:, :10]


def kernel(x, conv1_w, conv1_b, conv2_w, conv2_b, conv3_w, conv3_b,
           fc1_w, fc1_b, fc2_w, fc2_b, bn_gamma, bn_beta, bn_mean, bn_var):
    return _forward(x, conv1_w, conv1_b, conv2_w, conv2_b, conv3_w, conv3_b,
                    fc1_w, fc1_b, fc2_w, fc2_b, bn_gamma, bn_beta,
                    bn_mean, bn_var)


# speedup vs baseline: 1.1793x; 1.0161x over previous
import numpy as np
import jax
import jax.numpy as jnp
from jax.experimental import pallas as pl
from jax.experimental.pallas import tpu as pltpu


def _toeplitz(w_hwio, win, order, ncols, col0):
    k, _, cin, cout = w_hwio.shape
    nwo = len(order)
    s = np.zeros((k, win, nwo), np.float32)
    for kw in range(k):
        for col, wo in enumerate(order):
            s[kw, wo + kw, col] = 1.0
    t = jnp.einsum("kxw,hkio->hxiwo", jnp.asarray(s), w_hwio)
    t = t.reshape(k, win * cin, nwo * cout)
    return jnp.zeros((k, win * cin, ncols),
                     jnp.float32).at[:, :, col0:col0 + nwo * cout].set(t)


def _dot(lhs, rhs):
    return jax.lax.dot_general(lhs, rhs, (((lhs.ndim - 1,), (0,)), ((), ())),
                               preferred_element_type=jnp.float32)


def _fused_kernel(x_ref, t1a_ref, t1b_ref, t2_ref, c2b_ref,
                  t3_ref, c3b_ref, w1_ref, b1_ref, g_ref, s_ref,
                  w2_ref, b2_ref, o_ref):
    x = x_ref[...]
    xa, xb = x[:, 0:6, :], x[:, 1:7, :]

    act1 = [jnp.maximum(_dot(xa, t1a_ref[c]) + _dot(xb, t1b_ref[c]), 0.0)
             for c in range(4)]

    a2 = [
        jnp.pad(act1[2], ((0, 0), (1, 0), (0, 0))),
        jnp.pad(act1[3], ((0, 0), (1, 0), (0, 0))),
        jnp.pad(act1[0], ((0, 0), (0, 1), (0, 0))),
        jnp.pad(act1[1], ((0, 0), (0, 1), (0, 0))),
    ]

    act2 = []
    for p in range(2):
        hacc = []
        for hh in range(2):
            acc = None
            for kh in range(5):
                r = 2 * p + hh + kh
                q, s0 = r % 4, r // 4
                h = _dot(a2[q][:, s0:s0 + 6, :], t2_ref[kh])
                acc = h if acc is None else acc + h
            hacc.append(acc)
        z = jnp.maximum(jnp.maximum(hacc[0], hacc[1]) + c2b_ref[...], 0.0)
        act2.append(jnp.maximum(z[:, :, :96], z[:, :, 96:]))

    hacc3 = []
    for hh in range(2):
        acc = None
        for kh in range(5):
            r = hh + kh
            q, s0 = r % 2, r // 2
            h = _dot(act2[q][:, s0:s0 + 4, :], t3_ref[kh])
            acc = h if acc is None else acc + h
        hacc3.append(acc)
    z3 = jnp.maximum(jnp.maximum(hacc3[0], hacc3[1]) + c3b_ref[...], 0.0)
    act3 = jnp.maximum(z3[:, :, :64], z3[:, :, 64:])

    acc = None
    for h4 in range(4):
        h = _dot(act3[:, h4, :], w1_ref[h4])
        acc = h if acc is None else acc + h
    h = jnp.maximum(acc + b1_ref[...], 0.0)
    h = jnp.maximum(h * g_ref[...] + s_ref[...], 0.0)
    zz = _dot(h, w2_ref[...]) + b2_ref[...]
    m = jnp.max(zz, axis=1, keepdims=True)
    sz = zz - m
    lse = jnp.log(jnp.sum(jnp.exp(sz), axis=1, keepdims=True))
    o_ref[...] = (sz - lse).astype(o_ref.dtype)


@jax.jit
def _forward(x, conv1_w, conv1_b, conv2_w, conv2_b, conv3_w, conv3_b,
             fc1_w, fc1_b, fc2_w, fc2_b, bn_gamma, bn_beta, bn_mean, bn_var):
    b = x.shape[0]
    x7 = jnp.concatenate([x.reshape(b, 7, 112),
                          jnp.ones((b, 7, 1), jnp.float32)], axis=2)

    sa = np.zeros((4, 5, 5, 113, 24), np.float32)
    sb = np.zeros_like(sa)
    for c in range(4):
        for kh in range(5):
            tgt, q = (sa, c + kh) if c + kh < 4 else (sb, c + kh - 4)
            for kw in range(5):
                for wo in range(24):
                    tgt[c, kh, kw, q * 28 + wo + kw, wo] = 1.0
    w1sq = conv1_w[:, :, 0, :]
    t1a = jnp.einsum("chkxw,hko->cxwo", jnp.asarray(sa),
                     w1sq).reshape(4, 113, 192)
    t1b = jnp.einsum("chkxw,hko->cxwo", jnp.asarray(sb),
                     w1sq).reshape(4, 113, 192)
    t1a = t1a.at[:, 112, :].set(jnp.tile(conv1_b, 24))
    zpad16 = jnp.zeros((4, 113, 16), jnp.float32)
    t1a = jnp.concatenate([zpad16, t1a, zpad16], axis=2)
    t1b = jnp.concatenate([zpad16, t1b, zpad16], axis=2)

    ev24 = list(range(0, 24, 2))
    od24 = list(range(1, 24, 2))
    t2 = (_toeplitz(conv2_w, 28, ev24, 192, 0)
          + _toeplitz(conv2_w, 28, od24, 192, 96))
    t3 = (_toeplitz(conv3_w, 12, [0, 2, 4, 6], 128, 0)
          + _toeplitz(conv3_w, 12, [1, 3, 5, 7], 128, 64))

    c2b = jnp.concatenate([jnp.tile(conv2_b, 12)] * 2).reshape(1, 1, 192)
    c3b = jnp.concatenate([jnp.tile(conv3_b, 4)] * 2).reshape(1, 1, 128)

    w1 = fc1_w.reshape(16, 4, 4, 64).transpose(1, 2, 0, 3).reshape(4, 64, 64)
    scale = bn_gamma * jax.lax.rsqrt(bn_var + 1e-5)
    shift = bn_beta - bn_mean * scale
    w2p = jnp.zeros((64, 128), jnp.float32).at[:, :10].set(fc2_w)
    b2p = jnp.full((1, 128), -1e30, jnp.float32).at[0, :10].set(fc2_b)

    tb = 256 if b % 256 == 0 else b
    flops = 2 * b * (6 * 2 * 4 * 112 * 192 + 6 * 4 * 5 * 224 * 192
                     + 4 * 2 * 5 * 96 * 128 + 4 * 64 * 64 + 64 * 128)
    out = pl.pallas_call(
        _fused_kernel,
        out_shape=jax.ShapeDtypeStruct((b, 128), jnp.float32),
        grid_spec=pltpu.PrefetchScalarGridSpec(
            num_scalar_prefetch=0,
            grid=(b // tb,),
            in_specs=[
                pl.BlockSpec((tb, 7, 113), lambda i: (i, 0, 0)),
                pl.BlockSpec((4, 113, 224), lambda i: (0, 0, 0)),
                pl.BlockSpec((4, 113, 224), lambda i: (0, 0, 0)),
                pl.BlockSpec((5, 224, 192), lambda i: (0, 0, 0)),
                pl.BlockSpec((1, 1, 192), lambda i: (0, 0, 0)),
                pl.BlockSpec((5, 96, 128), lambda i: (0, 0, 0)),
                pl.BlockSpec((1, 1, 128), lambda i: (0, 0, 0)),
                pl.BlockSpec((4, 64, 64), lambda i: (0, 0, 0)),
                pl.BlockSpec((1, 64), lambda i: (0, 0)),
                pl.BlockSpec((1, 64), lambda i: (0, 0)),
                pl.BlockSpec((1, 64), lambda i: (0, 0)),
                pl.BlockSpec((64, 128), lambda i: (0, 0)),
                pl.BlockSpec((1, 128), lambda i: (0, 0)),
            ],
            out_specs=pl.BlockSpec((tb, 128), lambda i: (i, 0)),
        ),
        compiler_params=pltpu.CompilerParams(
            dimension_semantics=("parallel",)),
        cost_estimate=pl.CostEstimate(
            flops=int(flops), transcendentals=int(b * 128),
            bytes_accessed=int(x.size * 4 + b * 128 * 4)),
    )(x7, t1a, t1b, t2, c2b, t3, c3b, w1,
      fc1_b.reshape(1, 64), scale.reshape(1, 64), shift.reshape(1, 64),
      w2p, b2p)
    return out[:, :10]


def kernel(x, conv1_w, conv1_b, conv2_w, conv2_b, conv3_w, conv3_b,
           fc1_w, fc1_b, fc2_w, fc2_b, bn_gamma, bn_beta, bn_mean, bn_var):
    return _forward(x, conv1_w, conv1_b, conv2_w, conv2_b, conv3_w, conv3_b,
                    fc1_w, fc1_b, fc2_w, fc2_b, bn_gamma, bn_beta,
                    bn_mean, bn_var)


# speedup vs baseline: 1.2320x; 1.0448x over previous
import numpy as np
import jax
import jax.numpy as jnp
from jax.experimental import pallas as pl
from jax.experimental.pallas import tpu as pltpu


def _toeplitz(w_hwio, win, order):
    k, _, cin, cout = w_hwio.shape
    nwo = len(order)
    s = np.zeros((k, win, nwo), np.float32)
    for kw in range(k):
        for col, wo in enumerate(order):
            s[kw, wo + kw, col] = 1.0
    t = jnp.einsum("kxw,hkio->hxiwo", jnp.asarray(s), w_hwio)
    return t.reshape(k, win * cin, nwo * cout)


def _dot(lhs, rhs):
    return jax.lax.dot_general(lhs, rhs, (((lhs.ndim - 1,), (0,)), ((), ())),
                               preferred_element_type=jnp.float32)


def _fused_kernel(x_ref, t1a_ref, t1b_ref, c1b_ref, t2_ref, c2b_ref,
                  t3_ref, c3b_ref, w1_ref, b1_ref, g_ref, s_ref,
                  w2_ref, b2_ref, o_ref):
    x = x_ref[...]
    xa, xb = x[:, 0:6, :], x[:, 1:7, :]

    act1 = []
    for c in range(4):
        acc = _dot(xa, t1a_ref[c]) + _dot(xb, t1b_ref[c])
        act1.append(jnp.maximum(acc + c1b_ref[...], 0.0))

    a2 = [
        jnp.pad(act1[2], ((0, 0), (1, 0), (16, 16))),
        jnp.pad(act1[3], ((0, 0), (1, 0), (16, 16))),
        jnp.pad(act1[0], ((0, 0), (0, 1), (16, 16))),
        jnp.pad(act1[1], ((0, 0), (0, 1), (16, 16))),
    ]

    act2 = []
    for p in range(2):
        hacc = []
        for hh in range(2):
            acc = None
            for kh in range(5):
                r = 2 * p + hh + kh
                q, s0 = r % 4, r // 4
                h = _dot(a2[q][:, s0:s0 + 6, :], t2_ref[kh])
                acc = h if acc is None else acc + h
            hacc.append(acc)
        z = jnp.maximum(jnp.maximum(hacc[0], hacc[1]) + c2b_ref[...], 0.0)
        act2.append(jnp.maximum(z[:, :, :96], z[:, :, 96:]))

    hacc3 = []
    for hh in range(2):
        acc = None
        for kh in range(5):
            r = hh + kh
            q, s0 = r % 2, r // 2
            h = _dot(act2[q][:, s0:s0 + 4, :], t3_ref[kh])
            acc = h if acc is None else acc + h
        hacc3.append(acc)
    z3 = jnp.maximum(jnp.maximum(hacc3[0], hacc3[1]) + c3b_ref[...], 0.0)
    act3 = jnp.maximum(z3[:, :, :64], z3[:, :, 64:])

    acc = None
    for h4 in range(4):
        h = _dot(act3[:, h4, :], w1_ref[h4])
        acc = h if acc is None else acc + h
    h = jnp.maximum(acc + b1_ref[...], 0.0)
    h = jnp.maximum(h * g_ref[...] + s_ref[...], 0.0)
    zz = _dot(h, w2_ref[...]) + b2_ref[...]
    m = jnp.max(zz, axis=1, keepdims=True)
    sz = zz - m
    lse = jnp.log(jnp.sum(jnp.exp(sz), axis=1, keepdims=True))
    o_ref[...] = (sz - lse).astype(o_ref.dtype)


@jax.jit
def _forward(x, conv1_w, conv1_b, conv2_w, conv2_b, conv3_w, conv3_b,
             fc1_w, fc1_b, fc2_w, fc2_b, bn_gamma, bn_beta, bn_mean, bn_var):
    b = x.shape[0]
    x7 = x.reshape(b, 7, 112)

    sa = np.zeros((4, 5, 5, 112, 24), np.float32)
    sb = np.zeros_like(sa)
    for c in range(4):
        for kh in range(5):
            tgt, q = (sa, c + kh) if c + kh < 4 else (sb, c + kh - 4)
            for kw in range(5):
                for wo in range(24):
                    tgt[c, kh, kw, q * 28 + wo + kw, wo] = 1.0
    w1sq = conv1_w[:, :, 0, :]
    t1a = jnp.einsum("chkxw,hko->cxwo", jnp.asarray(sa),
                     w1sq).reshape(4, 112, 192)
    t1b = jnp.einsum("chkxw,hko->cxwo", jnp.asarray(sb),
                     w1sq).reshape(4, 112, 192)

    even_odd_24 = [w for par in (0, 1) for w in range(par, 24, 2)]
    even_odd_8 = [w for par in (0, 1) for w in range(par, 8, 2)]
    t2 = _toeplitz(conv2_w, 28, even_odd_24)
    t3 = _toeplitz(conv3_w, 12, even_odd_8)

    c1b = jnp.tile(conv1_b, 24).reshape(1, 1, 192)
    c2b = jnp.tile(conv2_b, 24).reshape(1, 1, 192)
    c3b = jnp.tile(conv3_b, 8).reshape(1, 1, 128)

    w1 = fc1_w.reshape(16, 4, 4, 64).transpose(1, 2, 0, 3).reshape(4, 64, 64)
    scale = bn_gamma * jax.lax.rsqrt(bn_var + 1e-5)
    shift = bn_beta - bn_mean * scale
    w2p = jnp.zeros((64, 128), jnp.float32).at[:, :10].set(fc2_w)
    b2p = jnp.full((1, 128), -1e30, jnp.float32).at[0, :10].set(fc2_b)

    tb = 256 if b % 256 == 0 else b
    flops = 2 * b * (6 * 2 * 4 * 112 * 192 + 6 * 4 * 5 * 224 * 192
                     + 4 * 2 * 5 * 96 * 128 + 4 * 64 * 64 + 64 * 128)
    out = pl.pallas_call(
        _fused_kernel,
        out_shape=jax.ShapeDtypeStruct((b, 128), jnp.float32),
        grid_spec=pltpu.PrefetchScalarGridSpec(
            num_scalar_prefetch=0,
            grid=(b // tb,),
            in_specs=[
                pl.BlockSpec((tb, 7, 112), lambda i: (i, 0, 0)),
                pl.BlockSpec((4, 112, 192), lambda i: (0, 0, 0)),
                pl.BlockSpec((4, 112, 192), lambda i: (0, 0, 0)),
                pl.BlockSpec((1, 1, 192), lambda i: (0, 0, 0)),
                pl.BlockSpec((5, 224, 192), lambda i: (0, 0, 0)),
                pl.BlockSpec((1, 1, 192), lambda i: (0, 0, 0)),
                pl.BlockSpec((5, 96, 128), lambda i: (0, 0, 0)),
                pl.BlockSpec((1, 1, 128), lambda i: (0, 0, 0)),
                pl.BlockSpec((4, 64, 64), lambda i: (0, 0, 0)),
                pl.BlockSpec((1, 64), lambda i: (0, 0)),
                pl.BlockSpec((1, 64), lambda i: (0, 0)),
                pl.BlockSpec((1, 64), lambda i: (0, 0)),
                pl.BlockSpec((64, 128), lambda i: (0, 0)),
                pl.BlockSpec((1, 128), lambda i: (0, 0)),
            ],
            out_specs=pl.BlockSpec((tb, 128), lambda i: (i, 0)),
        ),
        compiler_params=pltpu.CompilerParams(
            dimension_semantics=("parallel",)),
        cost_estimate=pl.CostEstimate(
            flops=int(flops), transcendentals=int(b * 128),
            bytes_accessed=int(x.size * 4 + b * 128 * 4)),
    )(x7, t1a, t1b, c1b, t2, c2b, t3, c3b, w1,
      fc1_b.reshape(1, 64), scale.reshape(1, 64), shift.reshape(1, 64),
      w2p, b2p)
    return out[:, :10]


def kernel(x, conv1_w, conv1_b, conv2_w, conv2_b, conv3_w, conv3_b,
           fc1_w, fc1_b, fc2_w, fc2_b, bn_gamma, bn_beta, bn_mean, bn_var):
    return _forward(x, conv1_w, conv1_b, conv2_w, conv2_b, conv3_w, conv3_b,
                    fc1_w, fc1_b, fc2_w, fc2_b, bn_gamma, bn_beta,
                    bn_mean, bn_var)
